# baseline (speedup 1.0000x reference)
import functools

import jax
import jax.numpy as jnp
from jax import lax
from jax.experimental import pallas as pl
from jax.experimental.pallas import tpu as pltpu
from jax.experimental.pallas import tpu_sc as plsc

N = 10000
E = 320000
D = 128
H = 128
C = 10

NC = 2
NS = 16
NW = NC * NS
NP = 10240
CH = 128
EPW = NP
EP = NW * EPW
NB = EPW // CH
RPS = NP // NS

def _sc_pass_body(y_hbm, src_hbm, dst_hbm, out_hbm, sidx_v, didx_v, rows_v,
                  sems, acc_sh):
    c = lax.axis_index("c")
    s = lax.axis_index("s")
    w = s * NC + c

    def zfill(i, _):
        for jj in range(D // 16):
            rows_v[0, i, pl.ds(jj * 16, 16)] = jnp.zeros((16,), jnp.float32)
        return 0

    lax.fori_loop(0, CH, zfill, 0)

    for kk in range(5):
        pltpu.sync_copy(
            rows_v.at[0], acc_sh.at[pl.ds(s * RPS + kk * (RPS // 5), RPS // 5), :]
        )
    plsc.subcore_barrier()

    EXPC = 1
    nch = 2 * NB

    @pl.when(c == EXPC)
    def _():
        def prime(b):
            base = (s * nch + b) * CH
            pltpu.sync_copy(src_hbm.at[pl.ds(base, CH)], sidx_v.at[b])
            pltpu.sync_copy(dst_hbm.at[pl.ds(base, CH)], didx_v.at[b])
            pltpu.async_copy(y_hbm.at[sidx_v.at[b]], rows_v.at[b], sems.at[b])

        for b in range(2):
            prime(b)

        def body(i, _):
            j0 = i * 2
            for b in range(2):
                pltpu.make_async_copy(
                    y_hbm.at[sidx_v.at[b]], rows_v.at[b], sems.at[b]
                ).wait()
                pltpu.sync_copy(rows_v.at[b], acc_sh.at[didx_v.at[b]], add=True)

                @pl.when(j0 + 2 + b < nch)
                def _():
                    base = (s * nch + j0 + 2 + b) * CH
                    pltpu.sync_copy(src_hbm.at[pl.ds(base, CH)], sidx_v.at[b])
                    pltpu.sync_copy(dst_hbm.at[pl.ds(base, CH)], didx_v.at[b])
                    pltpu.async_copy(y_hbm.at[sidx_v.at[b]], rows_v.at[b], sems.at[b])

            return 0

        lax.fori_loop(0, nch // 2, body, 0)
    plsc.subcore_barrier()
    pltpu.sync_copy(
        acc_sh.at[pl.ds(s * RPS, RPS), :],
        out_hbm.at[c, pl.ds(s * RPS, RPS), :],
    )


@functools.cache
def _build_sc_kernels():
    mesh = plsc.VectorSubcoreMesh(
        core_axis_name="c", subcore_axis_name="s", num_cores=NC, num_subcores=NS
    )
    sc_pass = pl.kernel(
        _sc_pass_body,
        mesh=mesh,
        out_type=jax.ShapeDtypeStruct((NC, NP, D), jnp.float32),
        scratch_types=[
            pltpu.VMEM((2, CH), jnp.int32),
            pltpu.VMEM((2, CH), jnp.int32),
            pltpu.VMEM((2, CH, D), jnp.float32),
            pltpu.SemaphoreType.DMA((2,)),
            pltpu.VMEM_SHARED((NP, D), jnp.float32),
        ],
    )
    return sc_pass


BR = 1000
NG = N // BR


def _tc_norm_body(pdeg_ref, x_ref, norm_ref, y0_ref):
    d2 = pdeg_ref[0] + pdeg_ref[1]
    d = d2[:, 0:1]
    nrm = lax.rsqrt(jnp.where(d > 0.0, d, 1.0))
    norm_ref[...] = nrm
    y0_ref[...] = x_ref[...] * nrm


_tc_norm = pl.pallas_call(
    _tc_norm_body,
    grid=(NG,),
    in_specs=[
        pl.BlockSpec((NC, BR, D), lambda i: (0, i, 0)),
        pl.BlockSpec((BR, D), lambda i: (i, 0)),
    ],
    out_specs=[
        pl.BlockSpec((BR, 1), lambda i: (i, 0)),
        pl.BlockSpec((BR, D), lambda i: (i, 0)),
    ],
    out_shape=[
        jax.ShapeDtypeStruct((N, 1), jnp.float32),
        jax.ShapeDtypeStruct((N, D), jnp.float32),
    ],
)


def _tc_mid_body(p_ref, norm_ref, a_ref, y_ref):
    ssum = p_ref[0] + p_ref[1]
    nrm = norm_ref[...]
    a = ssum * nrm
    a_ref[...] = a
    y_ref[...] = a * nrm


_tc_mid = pl.pallas_call(
    _tc_mid_body,
    grid=(NG,),
    in_specs=[
        pl.BlockSpec((NC, BR, D), lambda i: (0, i, 0)),
        pl.BlockSpec((BR, 1), lambda i: (i, 0)),
    ],
    out_specs=[
        pl.BlockSpec((BR, D), lambda i: (i, 0)),
        pl.BlockSpec((BR, D), lambda i: (i, 0)),
    ],
    out_shape=[
        jax.ShapeDtypeStruct((N, D), jnp.float32),
        jax.ShapeDtypeStruct((N, D), jnp.float32),
    ],
)


def _tc_layer1_body(x_ref, a1_ref, a2_ref, w_ref, b_ref, norm_ref, h_ref, y_ref):
    acc = jnp.dot(x_ref[...], w_ref[0:D, :], preferred_element_type=jnp.float32)
    acc += jnp.dot(a1_ref[...], w_ref[D:2 * D, :], preferred_element_type=jnp.float32)
    acc += jnp.dot(a2_ref[...], w_ref[2 * D:3 * D, :], preferred_element_type=jnp.float32)
    h = jnp.maximum(acc + b_ref[...], 0.0)
    h_ref[...] = h
    y_ref[...] = h * norm_ref[...]


_tc_layer1 = pl.pallas_call(
    _tc_layer1_body,
    grid=(NG,),
    in_specs=[
        pl.BlockSpec((BR, D), lambda i: (i, 0)),
        pl.BlockSpec((BR, D), lambda i: (i, 0)),
        pl.BlockSpec((BR, D), lambda i: (i, 0)),
        pl.BlockSpec((3 * D, H), lambda i: (0, 0)),
        pl.BlockSpec((1, H), lambda i: (0, 0)),
        pl.BlockSpec((BR, 1), lambda i: (i, 0)),
    ],
    out_specs=[
        pl.BlockSpec((BR, H), lambda i: (i, 0)),
        pl.BlockSpec((BR, H), lambda i: (i, 0)),
    ],
    out_shape=[
        jax.ShapeDtypeStruct((N, H), jnp.float32),
        jax.ShapeDtypeStruct((N, H), jnp.float32),
    ],
)


def _tc_layer2_body(x_ref, a1_ref, a2_ref, w_ref, b_ref, wc_ref, bc_ref,
                    out_ref, mx_ref):
    i = pl.program_id(0)
    acc = jnp.dot(x_ref[...], w_ref[0:H, :], preferred_element_type=jnp.float32)
    acc += jnp.dot(a1_ref[...], w_ref[H:2 * H, :], preferred_element_type=jnp.float32)
    acc += jnp.dot(a2_ref[...], w_ref[2 * H:3 * H, :], preferred_element_type=jnp.float32)
    h = jnp.maximum(acc + b_ref[...], 0.0)
    m = jnp.max(h, axis=0, keepdims=True)

    @pl.when(i == 0)
    def _():
        mx_ref[...] = jnp.broadcast_to(m, (8, H))

    @pl.when(i > 0)
    def _():
        mx_ref[...] = jnp.maximum(mx_ref[...], m)

    @pl.when(i == NG - 1)
    def _():
        g = jnp.max(mx_ref[...], axis=0, keepdims=True)
        out_ref[...] = (
            jnp.dot(g, wc_ref[...], preferred_element_type=jnp.float32)
            + bc_ref[...]
        )


_tc_layer2 = pl.pallas_call(
    _tc_layer2_body,
    grid=(NG,),
    in_specs=[
        pl.BlockSpec((BR, H), lambda i: (i, 0)),
        pl.BlockSpec((BR, H), lambda i: (i, 0)),
        pl.BlockSpec((BR, H), lambda i: (i, 0)),
        pl.BlockSpec((3 * H, H), lambda i: (0, 0)),
        pl.BlockSpec((1, H), lambda i: (0, 0)),
        pl.BlockSpec((H, C), lambda i: (0, 0)),
        pl.BlockSpec((1, C), lambda i: (0, 0)),
    ],
    out_specs=pl.BlockSpec((1, C), lambda i: (0, 0)),
    out_shape=jax.ShapeDtypeStruct((1, C), jnp.float32),
    scratch_shapes=[pltpu.VMEM((8, H), jnp.float32)],
)


def kernel(x, edge_index, W1, b1, W2, b2, Wc, bc):
    npad = EP - E
    src = jnp.concatenate([edge_index[0], jnp.zeros((npad,), jnp.int32)])
    dst = jnp.concatenate([edge_index[1], jnp.full((npad,), N, jnp.int32)])
    b1r = b1.reshape(1, H)
    b2r = b2.reshape(1, H)
    bcr = bc.reshape(1, C)
    _sc_pass = _build_sc_kernels()

    ones_tab = jnp.ones((NP, D), jnp.float32)
    pdeg = _sc_pass(ones_tab, dst, dst)
    norm, y0 = _tc_norm(pdeg, x)

    p1 = _sc_pass(y0, src, dst)
    a1, y1 = _tc_mid(p1, norm)
    p2 = _sc_pass(y1, src, dst)
    a2, _ = _tc_mid(p2, norm)

    h1, yh1 = _tc_layer1(x, a1, a2, W1, b1r, norm)

    p3 = _sc_pass(yh1, src, dst)
    c1, yc1 = _tc_mid(p3, norm)
    p4 = _sc_pass(yc1, src, dst)
    c2, _ = _tc_mid(p4, norm)

    return _tc_layer2(h1, c1, c2, W2, b2r, Wc, bcr)


# speedup vs baseline: 3.9127x; 3.9127x over previous
import functools

import jax
import jax.numpy as jnp
from jax import lax
from jax.experimental import pallas as pl
from jax.experimental.pallas import tpu as pltpu
from jax.experimental.pallas import tpu_sc as plsc

N = 10000
E = 320000
D = 128
H = 128
C = 10

NC = 2
NS = 16
NW = NC * NS
NP = 10240
CH = 128
EPW = NP
EP = NW * EPW
NB = EPW // CH
RPS = NP // NS

def _sc_pass_body(y_hbm, src_hbm, dst_hbm, out_hbm, sidx_v, didx_v, rows_v,
                  sems, acc_sh):
    c = lax.axis_index("c")
    s = lax.axis_index("s")
    w = s * NC + c

    def zfill(i, _):
        for jj in range(D // 16):
            rows_v[0, i, pl.ds(jj * 16, 16)] = jnp.zeros((16,), jnp.float32)
        return 0

    lax.fori_loop(0, CH, zfill, 0)

    for kk in range(5):
        pltpu.sync_copy(
            rows_v.at[0], acc_sh.at[pl.ds(s * RPS + kk * (RPS // 5), RPS // 5), :]
        )
    plsc.subcore_barrier()

    def prime(b):
        base = w * EPW + b * CH
        pltpu.sync_copy(src_hbm.at[pl.ds(base, CH)], sidx_v.at[b])
        pltpu.sync_copy(dst_hbm.at[pl.ds(base, CH)], didx_v.at[b])
        pltpu.async_copy(y_hbm.at[sidx_v.at[b]], rows_v.at[b], sems.at[b])

    for b in range(2):
        prime(b)

    def body(i, _):
        j0 = i * 2
        for b in range(2):
            pltpu.make_async_copy(
                y_hbm.at[sidx_v.at[b]], rows_v.at[b], sems.at[b]
            ).wait()
            pltpu.sync_copy(rows_v.at[b], acc_sh.at[didx_v.at[b]], add=True)

            @pl.when(j0 + 2 + b < NB)
            def _():
                base = w * EPW + (j0 + 2 + b) * CH
                pltpu.sync_copy(src_hbm.at[pl.ds(base, CH)], sidx_v.at[b])
                pltpu.sync_copy(dst_hbm.at[pl.ds(base, CH)], didx_v.at[b])
                pltpu.async_copy(y_hbm.at[sidx_v.at[b]], rows_v.at[b], sems.at[b])

        return 0

    lax.fori_loop(0, NB // 2, body, 0)
    plsc.subcore_barrier()
    pltpu.sync_copy(
        acc_sh.at[pl.ds(s * RPS, RPS), :],
        out_hbm.at[c, pl.ds(s * RPS, RPS), :],
    )


@functools.cache
def _build_sc_kernels():
    mesh = plsc.VectorSubcoreMesh(
        core_axis_name="c", subcore_axis_name="s", num_cores=NC, num_subcores=NS
    )
    sc_pass = pl.kernel(
        _sc_pass_body,
        mesh=mesh,
        out_type=jax.ShapeDtypeStruct((NC, NP, D), jnp.float32),
        scratch_types=[
            pltpu.VMEM((2, CH), jnp.int32),
            pltpu.VMEM((2, CH), jnp.int32),
            pltpu.VMEM((2, CH, D), jnp.float32),
            pltpu.SemaphoreType.DMA((2,)),
            pltpu.VMEM_SHARED((NP, D), jnp.float32),
        ],
    )
    return sc_pass


BR = 1000
NG = N // BR


def _tc_norm_body(pdeg_ref, x_ref, norm_ref, y0_ref):
    d2 = pdeg_ref[0] + pdeg_ref[1]
    d = d2[:, 0:1]
    nrm = lax.rsqrt(jnp.where(d > 0.0, d, 1.0))
    norm_ref[...] = nrm
    y0_ref[...] = x_ref[...] * nrm


_tc_norm = pl.pallas_call(
    _tc_norm_body,
    grid=(NG,),
    in_specs=[
        pl.BlockSpec((NC, BR, D), lambda i: (0, i, 0)),
        pl.BlockSpec((BR, D), lambda i: (i, 0)),
    ],
    out_specs=[
        pl.BlockSpec((BR, 1), lambda i: (i, 0)),
        pl.BlockSpec((BR, D), lambda i: (i, 0)),
    ],
    out_shape=[
        jax.ShapeDtypeStruct((N, 1), jnp.float32),
        jax.ShapeDtypeStruct((N, D), jnp.float32),
    ],
)


def _tc_mid_body(p_ref, norm_ref, a_ref, y_ref):
    ssum = p_ref[0] + p_ref[1]
    nrm = norm_ref[...]
    a = ssum * nrm
    a_ref[...] = a
    y_ref[...] = a * nrm


_tc_mid = pl.pallas_call(
    _tc_mid_body,
    grid=(NG,),
    in_specs=[
        pl.BlockSpec((NC, BR, D), lambda i: (0, i, 0)),
        pl.BlockSpec((BR, 1), lambda i: (i, 0)),
    ],
    out_specs=[
        pl.BlockSpec((BR, D), lambda i: (i, 0)),
        pl.BlockSpec((BR, D), lambda i: (i, 0)),
    ],
    out_shape=[
        jax.ShapeDtypeStruct((N, D), jnp.float32),
        jax.ShapeDtypeStruct((N, D), jnp.float32),
    ],
)


def _tc_layer1_body(x_ref, a1_ref, a2_ref, w_ref, b_ref, norm_ref, h_ref, y_ref):
    acc = jnp.dot(x_ref[...], w_ref[0:D, :], preferred_element_type=jnp.float32)
    acc += jnp.dot(a1_ref[...], w_ref[D:2 * D, :], preferred_element_type=jnp.float32)
    acc += jnp.dot(a2_ref[...], w_ref[2 * D:3 * D, :], preferred_element_type=jnp.float32)
    h = jnp.maximum(acc + b_ref[...], 0.0)
    h_ref[...] = h
    y_ref[...] = h * norm_ref[...]


_tc_layer1 = pl.pallas_call(
    _tc_layer1_body,
    grid=(NG,),
    in_specs=[
        pl.BlockSpec((BR, D), lambda i: (i, 0)),
        pl.BlockSpec((BR, D), lambda i: (i, 0)),
        pl.BlockSpec((BR, D), lambda i: (i, 0)),
        pl.BlockSpec((3 * D, H), lambda i: (0, 0)),
        pl.BlockSpec((1, H), lambda i: (0, 0)),
        pl.BlockSpec((BR, 1), lambda i: (i, 0)),
    ],
    out_specs=[
        pl.BlockSpec((BR, H), lambda i: (i, 0)),
        pl.BlockSpec((BR, H), lambda i: (i, 0)),
    ],
    out_shape=[
        jax.ShapeDtypeStruct((N, H), jnp.float32),
        jax.ShapeDtypeStruct((N, H), jnp.float32),
    ],
)


def _tc_layer2_body(x_ref, a1_ref, a2_ref, w_ref, b_ref, wc_ref, bc_ref,
                    out_ref, mx_ref):
    i = pl.program_id(0)
    acc = jnp.dot(x_ref[...], w_ref[0:H, :], preferred_element_type=jnp.float32)
    acc += jnp.dot(a1_ref[...], w_ref[H:2 * H, :], preferred_element_type=jnp.float32)
    acc += jnp.dot(a2_ref[...], w_ref[2 * H:3 * H, :], preferred_element_type=jnp.float32)
    h = jnp.maximum(acc + b_ref[...], 0.0)
    m = jnp.max(h, axis=0, keepdims=True)

    @pl.when(i == 0)
    def _():
        mx_ref[...] = jnp.broadcast_to(m, (8, H))

    @pl.when(i > 0)
    def _():
        mx_ref[...] = jnp.maximum(mx_ref[...], m)

    @pl.when(i == NG - 1)
    def _():
        g = jnp.max(mx_ref[...], axis=0, keepdims=True)
        out_ref[...] = (
            jnp.dot(g, wc_ref[...], preferred_element_type=jnp.float32)
            + bc_ref[...]
        )


_tc_layer2 = pl.pallas_call(
    _tc_layer2_body,
    grid=(NG,),
    in_specs=[
        pl.BlockSpec((BR, H), lambda i: (i, 0)),
        pl.BlockSpec((BR, H), lambda i: (i, 0)),
        pl.BlockSpec((BR, H), lambda i: (i, 0)),
        pl.BlockSpec((3 * H, H), lambda i: (0, 0)),
        pl.BlockSpec((1, H), lambda i: (0, 0)),
        pl.BlockSpec((H, C), lambda i: (0, 0)),
        pl.BlockSpec((1, C), lambda i: (0, 0)),
    ],
    out_specs=pl.BlockSpec((1, C), lambda i: (0, 0)),
    out_shape=jax.ShapeDtypeStruct((1, C), jnp.float32),
    scratch_shapes=[pltpu.VMEM((8, H), jnp.float32)],
)


def kernel(x, edge_index, W1, b1, W2, b2, Wc, bc):
    npad = EP - E
    pr = jnp.arange(npad, dtype=jnp.int32)
    src = jnp.concatenate([edge_index[0], pr % N])
    dst = jnp.concatenate([edge_index[1], N + pr % (NP - N)])
    b1r = b1.reshape(1, H)
    b2r = b2.reshape(1, H)
    bcr = bc.reshape(1, C)
    _sc_pass = _build_sc_kernels()

    ones_tab = jnp.ones((NP, D), jnp.float32)
    pdeg = _sc_pass(ones_tab, dst, dst)
    norm, y0 = _tc_norm(pdeg, x)

    p1 = _sc_pass(y0, src, dst)
    a1, y1 = _tc_mid(p1, norm)
    p2 = _sc_pass(y1, src, dst)
    a2, _ = _tc_mid(p2, norm)

    h1, yh1 = _tc_layer1(x, a1, a2, W1, b1r, norm)

    p3 = _sc_pass(yh1, src, dst)
    c1, yc1 = _tc_mid(p3, norm)
    p4 = _sc_pass(yc1, src, dst)
    c2, _ = _tc_mid(p4, norm)

    return _tc_layer2(h1, c1, c2, W2, b2r, Wc, bcr)


# speedup vs baseline: 4.1554x; 1.0620x over previous
import functools

import jax
import jax.numpy as jnp
from jax import lax
from jax.experimental import pallas as pl
from jax.experimental.pallas import tpu as pltpu
from jax.experimental.pallas import tpu_sc as plsc

N = 10000
E = 320000
D = 128
H = 128
C = 10

NC = 2
NS = 16
NW = NC * NS
NP = 10240
CH = 128
EPW = NP
EP = NW * EPW
NB = EPW // CH
RPS = NP // NS

def _sc_pass_body(y_hbm, src_hbm, dst_hbm, out_hbm, sidx_v, didx_v, rows_v,
                  sems, acc_sh):
    c = lax.axis_index("c")
    s = lax.axis_index("s")
    w = s * NC + c

    def zfill(i, _):
        for jj in range(D // 16):
            rows_v[0, i, pl.ds(jj * 16, 16)] = jnp.zeros((16,), jnp.float32)
        return 0

    lax.fori_loop(0, CH, zfill, 0)

    for kk in range(5):
        pltpu.sync_copy(
            rows_v.at[0], acc_sh.at[pl.ds(s * RPS + kk * (RPS // 5), RPS // 5), :]
        )
    plsc.subcore_barrier()

    def prime(b):
        base = w * EPW + b * CH
        pltpu.sync_copy(src_hbm.at[pl.ds(base, CH)], sidx_v.at[b])
        pltpu.sync_copy(dst_hbm.at[pl.ds(base, CH)], didx_v.at[b])
        pltpu.async_copy(y_hbm.at[sidx_v.at[b]], rows_v.at[b], sems.at[b])

    for b in range(2):
        prime(b)

    def body(i, _):
        j0 = i * 2
        for b in range(2):
            pltpu.make_async_copy(
                y_hbm.at[sidx_v.at[b]], rows_v.at[b], sems.at[b]
            ).wait()
            pltpu.sync_copy(rows_v.at[b], acc_sh.at[didx_v.at[b]], add=True)

            @pl.when(j0 + 2 + b < NB)
            def _():
                base = w * EPW + (j0 + 2 + b) * CH
                pltpu.sync_copy(src_hbm.at[pl.ds(base, CH)], sidx_v.at[b])
                pltpu.sync_copy(dst_hbm.at[pl.ds(base, CH)], didx_v.at[b])
                pltpu.async_copy(y_hbm.at[sidx_v.at[b]], rows_v.at[b], sems.at[b])

        return 0

    lax.fori_loop(0, NB // 2, body, 0)
    plsc.subcore_barrier()
    pltpu.sync_copy(
        acc_sh.at[pl.ds(s * RPS, RPS), :],
        out_hbm.at[c, pl.ds(s * RPS, RPS), :],
    )


def _sc_deg_body(dst_hbm, out_hbm, didx_v, ones_v, acc_sh):
    c = lax.axis_index("c")
    s = lax.axis_index("s")
    w = s * NC + c

    def ofill(i, _):
        for jj in range(D // 16):
            ones_v[0, i, pl.ds(jj * 16, 16)] = jnp.zeros((16,), jnp.float32)
            ones_v[1, i, pl.ds(jj * 16, 16)] = jnp.ones((16,), jnp.float32)
        return 0

    lax.fori_loop(0, CH, ofill, 0)

    for kk in range(5):
        pltpu.sync_copy(
            ones_v.at[0], acc_sh.at[pl.ds(s * RPS + kk * (RPS // 5), RPS // 5), :]
        )
    plsc.subcore_barrier()

    def prime(b):
        base = w * EPW + b * CH
        pltpu.sync_copy(dst_hbm.at[pl.ds(base, CH)], didx_v.at[b])

    for b in range(2):
        prime(b)

    def body(i, _):
        j0 = i * 2
        for b in range(2):
            pltpu.sync_copy(ones_v.at[1], acc_sh.at[didx_v.at[b]], add=True)

            @pl.when(j0 + 2 + b < NB)
            def _():
                base = w * EPW + (j0 + 2 + b) * CH
                pltpu.sync_copy(dst_hbm.at[pl.ds(base, CH)], didx_v.at[b])

        return 0

    lax.fori_loop(0, NB // 2, body, 0)
    plsc.subcore_barrier()
    pltpu.sync_copy(
        acc_sh.at[pl.ds(s * RPS, RPS), :],
        out_hbm.at[c, pl.ds(s * RPS, RPS), :],
    )


@functools.cache
def _build_sc_kernels():
    mesh = plsc.VectorSubcoreMesh(
        core_axis_name="c", subcore_axis_name="s", num_cores=NC, num_subcores=NS
    )
    sc_pass = pl.kernel(
        _sc_pass_body,
        mesh=mesh,
        out_type=jax.ShapeDtypeStruct((NC, NP, D), jnp.float32),
        scratch_types=[
            pltpu.VMEM((2, CH), jnp.int32),
            pltpu.VMEM((2, CH), jnp.int32),
            pltpu.VMEM((2, CH, D), jnp.float32),
            pltpu.SemaphoreType.DMA((2,)),
            pltpu.VMEM_SHARED((NP, D), jnp.float32),
        ],
    )
    sc_deg = pl.kernel(
        _sc_deg_body,
        mesh=mesh,
        out_type=jax.ShapeDtypeStruct((NC, NP, D), jnp.float32),
        scratch_types=[
            pltpu.VMEM((2, CH), jnp.int32),
            pltpu.VMEM((2, CH, D), jnp.float32),
            pltpu.VMEM_SHARED((NP, D), jnp.float32),
        ],
    )
    return sc_pass, sc_deg


BR = 1000
NG = N // BR


def _tc_norm_body(pdeg_ref, x_ref, norm_ref, y0_ref):
    d2 = pdeg_ref[0] + pdeg_ref[1]
    d = d2[:, 0:1]
    nrm = lax.rsqrt(jnp.where(d > 0.0, d, 1.0))
    norm_ref[...] = nrm
    y0_ref[...] = x_ref[...] * nrm


_tc_norm = pl.pallas_call(
    _tc_norm_body,
    grid=(NG,),
    in_specs=[
        pl.BlockSpec((NC, BR, D), lambda i: (0, i, 0)),
        pl.BlockSpec((BR, D), lambda i: (i, 0)),
    ],
    out_specs=[
        pl.BlockSpec((BR, 1), lambda i: (i, 0)),
        pl.BlockSpec((BR, D), lambda i: (i, 0)),
    ],
    out_shape=[
        jax.ShapeDtypeStruct((N, 1), jnp.float32),
        jax.ShapeDtypeStruct((N, D), jnp.float32),
    ],
)


def _tc_mid_body(p_ref, norm_ref, a_ref, y_ref):
    ssum = p_ref[0] + p_ref[1]
    nrm = norm_ref[...]
    a = ssum * nrm
    a_ref[...] = a
    y_ref[...] = a * nrm


_tc_mid = pl.pallas_call(
    _tc_mid_body,
    grid=(NG,),
    in_specs=[
        pl.BlockSpec((NC, BR, D), lambda i: (0, i, 0)),
        pl.BlockSpec((BR, 1), lambda i: (i, 0)),
    ],
    out_specs=[
        pl.BlockSpec((BR, D), lambda i: (i, 0)),
        pl.BlockSpec((BR, D), lambda i: (i, 0)),
    ],
    out_shape=[
        jax.ShapeDtypeStruct((N, D), jnp.float32),
        jax.ShapeDtypeStruct((N, D), jnp.float32),
    ],
)


def _tc_layer1_body(x_ref, a1_ref, a2_ref, w_ref, b_ref, norm_ref, h_ref, y_ref):
    acc = jnp.dot(x_ref[...], w_ref[0:D, :], preferred_element_type=jnp.float32)
    acc += jnp.dot(a1_ref[...], w_ref[D:2 * D, :], preferred_element_type=jnp.float32)
    acc += jnp.dot(a2_ref[...], w_ref[2 * D:3 * D, :], preferred_element_type=jnp.float32)
    h = jnp.maximum(acc + b_ref[...], 0.0)
    h_ref[...] = h
    y_ref[...] = h * norm_ref[...]


_tc_layer1 = pl.pallas_call(
    _tc_layer1_body,
    grid=(NG,),
    in_specs=[
        pl.BlockSpec((BR, D), lambda i: (i, 0)),
        pl.BlockSpec((BR, D), lambda i: (i, 0)),
        pl.BlockSpec((BR, D), lambda i: (i, 0)),
        pl.BlockSpec((3 * D, H), lambda i: (0, 0)),
        pl.BlockSpec((1, H), lambda i: (0, 0)),
        pl.BlockSpec((BR, 1), lambda i: (i, 0)),
    ],
    out_specs=[
        pl.BlockSpec((BR, H), lambda i: (i, 0)),
        pl.BlockSpec((BR, H), lambda i: (i, 0)),
    ],
    out_shape=[
        jax.ShapeDtypeStruct((N, H), jnp.float32),
        jax.ShapeDtypeStruct((N, H), jnp.float32),
    ],
)


def _tc_layer2_body(x_ref, a1_ref, a2_ref, w_ref, b_ref, wc_ref, bc_ref,
                    out_ref, mx_ref):
    i = pl.program_id(0)
    acc = jnp.dot(x_ref[...], w_ref[0:H, :], preferred_element_type=jnp.float32)
    acc += jnp.dot(a1_ref[...], w_ref[H:2 * H, :], preferred_element_type=jnp.float32)
    acc += jnp.dot(a2_ref[...], w_ref[2 * H:3 * H, :], preferred_element_type=jnp.float32)
    h = jnp.maximum(acc + b_ref[...], 0.0)
    m = jnp.max(h, axis=0, keepdims=True)

    @pl.when(i == 0)
    def _():
        mx_ref[...] = jnp.broadcast_to(m, (8, H))

    @pl.when(i > 0)
    def _():
        mx_ref[...] = jnp.maximum(mx_ref[...], m)

    @pl.when(i == NG - 1)
    def _():
        g = jnp.max(mx_ref[...], axis=0, keepdims=True)
        out_ref[...] = (
            jnp.dot(g, wc_ref[...], preferred_element_type=jnp.float32)
            + bc_ref[...]
        )


_tc_layer2 = pl.pallas_call(
    _tc_layer2_body,
    grid=(NG,),
    in_specs=[
        pl.BlockSpec((BR, H), lambda i: (i, 0)),
        pl.BlockSpec((BR, H), lambda i: (i, 0)),
        pl.BlockSpec((BR, H), lambda i: (i, 0)),
        pl.BlockSpec((3 * H, H), lambda i: (0, 0)),
        pl.BlockSpec((1, H), lambda i: (0, 0)),
        pl.BlockSpec((H, C), lambda i: (0, 0)),
        pl.BlockSpec((1, C), lambda i: (0, 0)),
    ],
    out_specs=pl.BlockSpec((1, C), lambda i: (0, 0)),
    out_shape=jax.ShapeDtypeStruct((1, C), jnp.float32),
    scratch_shapes=[pltpu.VMEM((8, H), jnp.float32)],
)


def kernel(x, edge_index, W1, b1, W2, b2, Wc, bc):
    npad = EP - E
    pr = jnp.arange(npad, dtype=jnp.int32)
    src = jnp.concatenate([edge_index[0], pr % N])
    dst = jnp.concatenate([edge_index[1], N + pr % (NP - N)])
    b1r = b1.reshape(1, H)
    b2r = b2.reshape(1, H)
    bcr = bc.reshape(1, C)
    _sc_pass, _sc_deg = _build_sc_kernels()

    pdeg = _sc_deg(dst)
    norm, y0 = _tc_norm(pdeg, x)

    p1 = _sc_pass(y0, src, dst)
    a1, y1 = _tc_mid(p1, norm)
    p2 = _sc_pass(y1, src, dst)
    a2, _ = _tc_mid(p2, norm)

    h1, yh1 = _tc_layer1(x, a1, a2, W1, b1r, norm)

    p3 = _sc_pass(yh1, src, dst)
    c1, yc1 = _tc_mid(p3, norm)
    p4 = _sc_pass(yc1, src, dst)
    c2, _ = _tc_mid(p4, norm)

    return _tc_layer2(h1, c1, c2, W2, b2r, Wc, bcr)
